# baseline (speedup 1.0000x reference)
import functools

import jax
import jax.numpy as jnp
from jax import lax
from jax.experimental import pallas as pl
from jax.experimental.pallas import tpu as pltpu
from jax.experimental.pallas import tpu_sc as plsc

N = 10000
E = 320000
D = 128
H = 32
C = 10
G = 64

NTILES = 32
DEG_PER_TILE = E // NTILES
NGROUP = 8
NCHUNK = NTILES // NGROUP
FPG = H // NGROUP
EDGES_PER_CHUNK = E // NCHUNK
SUB = 8000
NSUB = EDGES_PER_CHUNK // SUB

_mesh = plsc.VectorSubcoreMesh(core_axis_name="c", subcore_axis_name="s")

_HIGH = lax.Precision.HIGHEST


def _dot(a, b, dims):
    return lax.dot_general(a, b, (dims, ((), ())),
                           precision=_HIGH, preferred_element_type=jnp.float32)


@functools.partial(
    pl.kernel,
    out_type=jax.ShapeDtypeStruct((NTILES, N), jnp.float32),
    mesh=_mesh,
    compiler_params=pltpu.CompilerParams(needs_layout_passes=False),
    scratch_types=[
        pltpu.VMEM((DEG_PER_TILE,), jnp.int32),
        pltpu.VMEM((DEG_PER_TILE,), jnp.float32),
        pltpu.VMEM((N,), jnp.float32),
    ],
)
def _deg_call(dst_hbm, ew_hbm, out_hbm, dst_v, ew_v, acc_v):
    tid = lax.axis_index("c") * 16 + lax.axis_index("s")
    base = tid * DEG_PER_TILE
    pltpu.sync_copy(dst_hbm.at[pl.ds(base, DEG_PER_TILE)], dst_v)
    pltpu.sync_copy(ew_hbm.at[pl.ds(base, DEG_PER_TILE)], ew_v)

    zero16 = jnp.zeros((16,), jnp.float32)

    @plsc.parallel_loop(0, N // 16, unroll=8)
    def zb(i):
        acc_v[pl.ds(i * 16, 16)] = zero16

    @plsc.parallel_loop(0, DEG_PER_TILE // 16, unroll=4)
    def eb(i):
        d = dst_v[pl.ds(i * 16, 16)]
        w = ew_v[pl.ds(i * 16, 16)]
        plsc.addupdate_scatter(acc_v, [d], w)

    pltpu.sync_copy(acc_v, out_hbm.at[tid])


@functools.partial(
    pl.kernel,
    out_type=jax.ShapeDtypeStruct((NCHUNK * H * N,), jnp.float32),
    mesh=_mesh,
    compiler_params=pltpu.CompilerParams(needs_layout_passes=False),
    scratch_types=[
        pltpu.VMEM((FPG * N,), jnp.float32),
        pltpu.VMEM((FPG * N,), jnp.float32),
        pltpu.VMEM((SUB,), jnp.int32),
        pltpu.VMEM((SUB,), jnp.int32),
        pltpu.VMEM((SUB,), jnp.float32),
    ],
)
def _agg_call(gt_hbm, src_hbm, dst_hbm, ew_hbm, out_hbm,
              gt_v, acc_v, src_v, dst_v, ew_v):
    tid = lax.axis_index("c") * 16 + lax.axis_index("s")
    grp = tid % NGROUP
    chunk = tid // NGROUP
    pltpu.sync_copy(gt_hbm.at[pl.ds(grp * FPG * N, FPG * N)], gt_v)

    zero16 = jnp.zeros((16,), jnp.float32)

    @plsc.parallel_loop(0, FPG * N // 16, unroll=8)
    def zb(i):
        acc_v[pl.ds(i * 16, 16)] = zero16

    for k in range(NSUB):
        base = chunk * EDGES_PER_CHUNK + k * SUB
        pltpu.sync_copy(src_hbm.at[pl.ds(base, SUB)], src_v)
        pltpu.sync_copy(dst_hbm.at[pl.ds(base, SUB)], dst_v)
        pltpu.sync_copy(ew_hbm.at[pl.ds(base, SUB)], ew_v)

        @plsc.parallel_loop(0, SUB // 16, unroll=8)
        def eb(i):
            s = src_v[pl.ds(i * 16, 16)]
            d = dst_v[pl.ds(i * 16, 16)]
            w = ew_v[pl.ds(i * 16, 16)]
            for f in range(FPG):
                si = s if f == 0 else s + f * N
                di = d if f == 0 else d + f * N
                v = plsc.load_gather(gt_v, [si])
                plsc.addupdate_scatter(acc_v, [di], v * w)

    out_off = (chunk * H + grp * FPG) * N
    pltpu.sync_copy(acc_v, out_hbm.at[pl.ds(out_off, FPG * N)])


def _pre_body(x_ref, w1_ref, degp_ref, g1t_ref, dinv_ref):
    deg = jnp.sum(degp_ref[...], axis=0, keepdims=True) + 1.0
    dinv = lax.rsqrt(deg)
    dinv_ref[...] = dinv
    h1t = _dot(w1_ref[...], x_ref[...], (((0,), (1,))))
    g1t_ref[...] = h1t * dinv


def _pre_call(x, W1, deg_part):
    return pl.pallas_call(
        _pre_body,
        out_shape=[
            jax.ShapeDtypeStruct((H, N), jnp.float32),
            jax.ShapeDtypeStruct((1, N), jnp.float32),
        ],
    )(x, W1, deg_part)


def _mid_body(part_ref, gt_ref, dinv_ref, b_ref, w_ref, gnext_ref):
    s = jnp.sum(part_ref[...], axis=0)
    dinv = dinv_ref[...]
    z = dinv * (s + gt_ref[...]) + b_ref[...]
    h = jnp.maximum(z, 0.0)
    a = _dot(w_ref[...], h, (((0,), (0,))))
    gnext_ref[...] = a * dinv


def _mid_call(part, gt, dinv, b, W):
    return pl.pallas_call(
        _mid_body,
        out_shape=jax.ShapeDtypeStruct((H, N), jnp.float32),
    )(part, gt, dinv, b, W)


def _post_body(part_ref, gt_ref, dinv_ref, b_ref, batch_ref, wl_ref, bl_ref,
               out_ref):
    s = jnp.sum(part_ref[...], axis=0)
    z = dinv_ref[...] * (s + gt_ref[...]) + b_ref[...]
    gid = lax.broadcasted_iota(jnp.int32, (G, N), 0)
    onehot = (batch_ref[...] == gid).astype(jnp.float32)
    sums = _dot(z, onehot, (((1,), (1,))))
    counts = jnp.maximum(jnp.sum(onehot, axis=1), 1.0)
    pooled = sums / counts[None, :]
    out_ref[...] = _dot(pooled, wl_ref[...], (((0,), (0,)))) + bl_ref[...]


def _post_call(part, gt, dinv, b, batch2d, Wl, bl):
    return pl.pallas_call(
        _post_body,
        out_shape=jax.ShapeDtypeStruct((G, C), jnp.float32),
    )(part, gt, dinv, b, batch2d, Wl, bl)


def kernel(x, edge_index, edge_weight, batch, W1, b1, W2, b2, W3, b3, Wl, bl):
    src = edge_index[0].astype(jnp.int32)
    dst = edge_index[1].astype(jnp.int32)
    ew = edge_weight.astype(jnp.float32)
    batch2d = batch.astype(jnp.int32).reshape(1, N)

    deg_part = _deg_call(dst, ew)
    g1t, dinv = _pre_call(x, W1, deg_part)

    p1 = _agg_call(g1t.reshape(-1), src, dst, ew).reshape(NCHUNK, H, N)
    g2t = _mid_call(p1, g1t, dinv, b1.reshape(H, 1), W2)

    p2 = _agg_call(g2t.reshape(-1), src, dst, ew).reshape(NCHUNK, H, N)
    g3t = _mid_call(p2, g2t, dinv, b2.reshape(H, 1), W3)

    p3 = _agg_call(g3t.reshape(-1), src, dst, ew).reshape(NCHUNK, H, N)
    return _post_call(p3, g3t, dinv, b3.reshape(H, 1), batch2d, Wl,
                      bl.reshape(1, C))


# speedup vs baseline: 1.0335x; 1.0335x over previous
import functools

import jax
import jax.numpy as jnp
from jax import lax
from jax.experimental import pallas as pl
from jax.experimental.pallas import tpu as pltpu
from jax.experimental.pallas import tpu_sc as plsc

N = 10000
E = 320000
D = 128
H = 32
C = 10
G = 64

NTILES = 32
DEG_PER_TILE = E // NTILES
NGROUP = 8
NCHUNK = NTILES // NGROUP
FPG = H // NGROUP
EDGES_PER_CHUNK = E // NCHUNK
SUB = 8000
NSUB = EDGES_PER_CHUNK // SUB

_mesh = plsc.VectorSubcoreMesh(core_axis_name="c", subcore_axis_name="s")

_HIGH = lax.Precision.HIGHEST


def _dot(a, b, dims):
    return lax.dot_general(a, b, (dims, ((), ())),
                           precision=_HIGH, preferred_element_type=jnp.float32)


@functools.partial(
    pl.kernel,
    out_type=jax.ShapeDtypeStruct((NTILES, N), jnp.float32),
    mesh=_mesh,
    compiler_params=pltpu.CompilerParams(needs_layout_passes=False),
    scratch_types=[
        pltpu.VMEM((DEG_PER_TILE,), jnp.int32),
        pltpu.VMEM((DEG_PER_TILE,), jnp.float32),
        pltpu.VMEM((N,), jnp.float32),
    ],
)
def _deg_call(dst_hbm, ew_hbm, out_hbm, dst_v, ew_v, acc_v):
    tid = lax.axis_index("c") * 16 + lax.axis_index("s")
    base = tid * DEG_PER_TILE
    pltpu.sync_copy(dst_hbm.at[pl.ds(base, DEG_PER_TILE)], dst_v)
    pltpu.sync_copy(ew_hbm.at[pl.ds(base, DEG_PER_TILE)], ew_v)

    zero16 = jnp.zeros((16,), jnp.float32)

    @plsc.parallel_loop(0, N // 16, unroll=8)
    def zb(i):
        acc_v[pl.ds(i * 16, 16)] = zero16

    @plsc.parallel_loop(0, DEG_PER_TILE // 16, unroll=4)
    def eb(i):
        d = dst_v[pl.ds(i * 16, 16)]
        w = ew_v[pl.ds(i * 16, 16)]
        plsc.addupdate_scatter(acc_v, [d], w)

    pltpu.sync_copy(acc_v, out_hbm.at[tid])


@functools.partial(
    pl.kernel,
    out_type=jax.ShapeDtypeStruct((NCHUNK * H * N,), jnp.float32),
    mesh=_mesh,
    compiler_params=pltpu.CompilerParams(needs_layout_passes=False),
    scratch_types=[
        pltpu.VMEM((FPG * N,), jnp.float32),
        pltpu.VMEM((FPG * N,), jnp.float32),
        pltpu.VMEM((SUB,), jnp.int32),
        pltpu.VMEM((SUB,), jnp.int32),
        pltpu.VMEM((SUB,), jnp.float32),
    ],
)
def _agg_call(gt_hbm, src_hbm, dst_hbm, ew_hbm, out_hbm,
              gt_v, acc_v, src_v, dst_v, ew_v):
    tid = lax.axis_index("c") * 16 + lax.axis_index("s")
    grp = tid % NGROUP
    chunk = tid // NGROUP
    pltpu.sync_copy(gt_hbm.at[pl.ds(grp * FPG * N, FPG * N)], gt_v)

    zero16 = jnp.zeros((16,), jnp.float32)

    @plsc.parallel_loop(0, FPG * N // 16, unroll=8)
    def zb(i):
        acc_v[pl.ds(i * 16, 16)] = zero16

    for k in range(NSUB):
        base = chunk * EDGES_PER_CHUNK + k * SUB
        pltpu.sync_copy(src_hbm.at[pl.ds(base, SUB)], src_v)
        pltpu.sync_copy(dst_hbm.at[pl.ds(base, SUB)], dst_v)
        pltpu.sync_copy(ew_hbm.at[pl.ds(base, SUB)], ew_v)

        @plsc.parallel_loop(0, SUB // 16, unroll=4)
        def eb(i):
            s = src_v[pl.ds(i * 16, 16)]
            d = dst_v[pl.ds(i * 16, 16)]
            w = ew_v[pl.ds(i * 16, 16)]
            for f in range(FPG):
                si = s if f == 0 else s + f * N
                di = d if f == 0 else d + f * N
                v = plsc.load_gather(gt_v, [si])
                plsc.addupdate_scatter(acc_v, [di], v * w)

    out_off = (chunk * H + grp * FPG) * N
    pltpu.sync_copy(acc_v, out_hbm.at[pl.ds(out_off, FPG * N)])


def _pre_body(x_ref, w1_ref, degp_ref, g1t_ref, dinv_ref):
    deg = jnp.sum(degp_ref[...], axis=0, keepdims=True) + 1.0
    dinv = lax.rsqrt(deg)
    dinv_ref[...] = dinv
    h1t = _dot(w1_ref[...], x_ref[...], (((0,), (1,))))
    g1t_ref[...] = h1t * dinv


def _pre_call(x, W1, deg_part):
    return pl.pallas_call(
        _pre_body,
        out_shape=[
            jax.ShapeDtypeStruct((H, N), jnp.float32),
            jax.ShapeDtypeStruct((1, N), jnp.float32),
        ],
    )(x, W1, deg_part)


def _mid_body(part_ref, gt_ref, dinv_ref, b_ref, w_ref, gnext_ref):
    s = jnp.sum(part_ref[...], axis=0)
    dinv = dinv_ref[...]
    z = dinv * (s + gt_ref[...]) + b_ref[...]
    h = jnp.maximum(z, 0.0)
    a = _dot(w_ref[...], h, (((0,), (0,))))
    gnext_ref[...] = a * dinv


def _mid_call(part, gt, dinv, b, W):
    return pl.pallas_call(
        _mid_body,
        out_shape=jax.ShapeDtypeStruct((H, N), jnp.float32),
    )(part, gt, dinv, b, W)


def _post_body(part_ref, gt_ref, dinv_ref, b_ref, batch_ref, wl_ref, bl_ref,
               out_ref):
    s = jnp.sum(part_ref[...], axis=0)
    z = dinv_ref[...] * (s + gt_ref[...]) + b_ref[...]
    gid = lax.broadcasted_iota(jnp.int32, (G, N), 0)
    onehot = (batch_ref[...] == gid).astype(jnp.float32)
    sums = _dot(z, onehot, (((1,), (1,))))
    counts = jnp.maximum(jnp.sum(onehot, axis=1), 1.0)
    pooled = sums / counts[None, :]
    out_ref[...] = _dot(pooled, wl_ref[...], (((0,), (0,)))) + bl_ref[...]


def _post_call(part, gt, dinv, b, batch2d, Wl, bl):
    return pl.pallas_call(
        _post_body,
        out_shape=jax.ShapeDtypeStruct((G, C), jnp.float32),
    )(part, gt, dinv, b, batch2d, Wl, bl)


def kernel(x, edge_index, edge_weight, batch, W1, b1, W2, b2, W3, b3, Wl, bl):
    src = edge_index[0].astype(jnp.int32)
    dst = edge_index[1].astype(jnp.int32)
    ew = edge_weight.astype(jnp.float32)
    batch2d = batch.astype(jnp.int32).reshape(1, N)

    deg_part = _deg_call(dst, ew)
    g1t, dinv = _pre_call(x, W1, deg_part)

    p1 = _agg_call(g1t.reshape(-1), src, dst, ew).reshape(NCHUNK, H, N)
    g2t = _mid_call(p1, g1t, dinv, b1.reshape(H, 1), W2)

    p2 = _agg_call(g2t.reshape(-1), src, dst, ew).reshape(NCHUNK, H, N)
    g3t = _mid_call(p2, g2t, dinv, b2.reshape(H, 1), W3)

    p3 = _agg_call(g3t.reshape(-1), src, dst, ew).reshape(NCHUNK, H, N)
    return _post_call(p3, g3t, dinv, b3.reshape(H, 1), batch2d, Wl,
                      bl.reshape(1, C))


# speedup vs baseline: 1.3697x; 1.3253x over previous
import functools

import jax
import jax.numpy as jnp
from jax import lax
from jax.experimental import pallas as pl
from jax.experimental.pallas import tpu as pltpu
from jax.experimental.pallas import tpu_sc as plsc

N = 10000
E = 320000
D = 128
H = 32
C = 10
G = 64

NTILES = 32
DEG_PER_TILE = E // NTILES
NGROUP = 8
NCHUNK = NTILES // NGROUP
FPG = H // NGROUP
EDGES_PER_CHUNK = E // NCHUNK
SUB = 8000
NSUB = EDGES_PER_CHUNK // SUB

_mesh = plsc.VectorSubcoreMesh(core_axis_name="c", subcore_axis_name="s")

_HIGH = lax.Precision.HIGHEST


def _dot(a, b, dims):
    return lax.dot_general(a, b, (dims, ((), ())),
                           precision=_HIGH, preferred_element_type=jnp.float32)


@functools.partial(
    pl.kernel,
    out_type=jax.ShapeDtypeStruct((NTILES, N), jnp.float32),
    mesh=_mesh,
    compiler_params=pltpu.CompilerParams(needs_layout_passes=False),
    scratch_types=[
        pltpu.VMEM((DEG_PER_TILE,), jnp.int32),
        pltpu.VMEM((DEG_PER_TILE,), jnp.float32),
        pltpu.VMEM((N,), jnp.float32),
    ],
)
def _deg_call(dst_hbm, ew_hbm, out_hbm, dst_v, ew_v, acc_v):
    tid = lax.axis_index("c") * 16 + lax.axis_index("s")
    base = tid * DEG_PER_TILE
    pltpu.sync_copy(dst_hbm.at[pl.ds(base, DEG_PER_TILE)], dst_v)
    pltpu.sync_copy(ew_hbm.at[pl.ds(base, DEG_PER_TILE)], ew_v)

    zero16 = jnp.zeros((16,), jnp.float32)

    @plsc.parallel_loop(0, N // 16, unroll=8)
    def zb(i):
        acc_v[pl.ds(i * 16, 16)] = zero16

    @plsc.parallel_loop(0, DEG_PER_TILE // 16, unroll=4)
    def eb(i):
        d = dst_v[pl.ds(i * 16, 16)]
        w = ew_v[pl.ds(i * 16, 16)]
        plsc.addupdate_scatter(acc_v, [d], w)

    pltpu.sync_copy(acc_v, out_hbm.at[tid])


@functools.partial(
    pl.kernel,
    out_type=jax.ShapeDtypeStruct((NCHUNK * H * N,), jnp.float32),
    mesh=_mesh,
    compiler_params=pltpu.CompilerParams(needs_layout_passes=False),
    scratch_types=[
        pltpu.VMEM((FPG * N,), jnp.float32),
        pltpu.VMEM((FPG * N,), jnp.float32),
        pltpu.VMEM((2 * SUB,), jnp.int32),
        pltpu.VMEM((2 * SUB,), jnp.int32),
        pltpu.VMEM((2 * SUB,), jnp.float32),
        pltpu.SemaphoreType.DMA,
        pltpu.SemaphoreType.DMA,
        pltpu.SemaphoreType.DMA,
    ],
)
def _agg_call(gt_hbm, src_hbm, dst_hbm, ew_hbm, out_hbm,
              gt_v, acc_v, src_v, dst_v, ew_v, gsem, sem0, sem1):
    tid = lax.axis_index("c") * 16 + lax.axis_index("s")
    grp = tid % NGROUP
    chunk = tid // NGROUP

    gt_h = pltpu.async_copy(gt_hbm.at[pl.ds(grp * FPG * N, FPG * N)], gt_v,
                            gsem)

    def issue(k):
        j = k % 2
        base = chunk * EDGES_PER_CHUNK + k * SUB
        sem = sem0 if j == 0 else sem1
        off = j * SUB
        return (
            pltpu.async_copy(src_hbm.at[pl.ds(base, SUB)],
                             src_v.at[pl.ds(off, SUB)], sem),
            pltpu.async_copy(dst_hbm.at[pl.ds(base, SUB)],
                             dst_v.at[pl.ds(off, SUB)], sem),
            pltpu.async_copy(ew_hbm.at[pl.ds(base, SUB)],
                             ew_v.at[pl.ds(off, SUB)], sem),
        )

    pending = [issue(0), None]

    zero16 = jnp.zeros((16,), jnp.float32)

    @plsc.parallel_loop(0, FPG * N // 16, unroll=8)
    def zb(i):
        acc_v[pl.ds(i * 16, 16)] = zero16

    gt_h.wait()

    for k in range(NSUB):
        j = k % 2
        if k + 1 < NSUB:
            pending[(k + 1) % 2] = issue(k + 1)
        for h in pending[j]:
            h.wait()
        off = j * SUB

        @plsc.parallel_loop(0, SUB // 16, unroll=4)
        def eb(i):
            s = src_v[pl.ds(off + i * 16, 16)]
            d = dst_v[pl.ds(off + i * 16, 16)]
            w = ew_v[pl.ds(off + i * 16, 16)]
            for f in range(FPG):
                si = s if f == 0 else s + f * N
                di = d if f == 0 else d + f * N
                v = plsc.load_gather(gt_v, [si])
                plsc.addupdate_scatter(acc_v, [di], v * w)

    out_off = (chunk * H + grp * FPG) * N
    pltpu.sync_copy(acc_v, out_hbm.at[pl.ds(out_off, FPG * N)])


def _pre_body(x_ref, w1_ref, degp_ref, g1t_ref, dinv_ref):
    deg = jnp.sum(degp_ref[...], axis=0, keepdims=True) + 1.0
    dinv = lax.rsqrt(deg)
    dinv_ref[...] = dinv
    h1t = _dot(w1_ref[...], x_ref[...], (((0,), (1,))))
    g1t_ref[...] = h1t * dinv


def _pre_call(x, W1, deg_part):
    return pl.pallas_call(
        _pre_body,
        out_shape=[
            jax.ShapeDtypeStruct((H, N), jnp.float32),
            jax.ShapeDtypeStruct((1, N), jnp.float32),
        ],
    )(x, W1, deg_part)


def _mid_body(part_ref, gt_ref, dinv_ref, b_ref, w_ref, gnext_ref):
    s = jnp.sum(part_ref[...], axis=0)
    dinv = dinv_ref[...]
    z = dinv * (s + gt_ref[...]) + b_ref[...]
    h = jnp.maximum(z, 0.0)
    a = _dot(w_ref[...], h, (((0,), (0,))))
    gnext_ref[...] = a * dinv


def _mid_call(part, gt, dinv, b, W):
    return pl.pallas_call(
        _mid_body,
        out_shape=jax.ShapeDtypeStruct((H, N), jnp.float32),
    )(part, gt, dinv, b, W)


def _post_body(part_ref, gt_ref, dinv_ref, b_ref, batch_ref, wl_ref, bl_ref,
               out_ref):
    s = jnp.sum(part_ref[...], axis=0)
    z = dinv_ref[...] * (s + gt_ref[...]) + b_ref[...]
    gid = lax.broadcasted_iota(jnp.int32, (G, N), 0)
    onehot = (batch_ref[...] == gid).astype(jnp.float32)
    sums = _dot(z, onehot, (((1,), (1,))))
    counts = jnp.maximum(jnp.sum(onehot, axis=1), 1.0)
    pooled = sums / counts[None, :]
    out_ref[...] = _dot(pooled, wl_ref[...], (((0,), (0,)))) + bl_ref[...]


def _post_call(part, gt, dinv, b, batch2d, Wl, bl):
    return pl.pallas_call(
        _post_body,
        out_shape=jax.ShapeDtypeStruct((G, C), jnp.float32),
    )(part, gt, dinv, b, batch2d, Wl, bl)


def kernel(x, edge_index, edge_weight, batch, W1, b1, W2, b2, W3, b3, Wl, bl):
    src = edge_index[0].astype(jnp.int32)
    dst = edge_index[1].astype(jnp.int32)
    ew = edge_weight.astype(jnp.float32)
    batch2d = batch.astype(jnp.int32).reshape(1, N)

    deg_part = _deg_call(dst, ew)
    g1t, dinv = _pre_call(x, W1, deg_part)

    p1 = _agg_call(g1t.reshape(-1), src, dst, ew).reshape(NCHUNK, H, N)
    g2t = _mid_call(p1, g1t, dinv, b1.reshape(H, 1), W2)

    p2 = _agg_call(g2t.reshape(-1), src, dst, ew).reshape(NCHUNK, H, N)
    g3t = _mid_call(p2, g2t, dinv, b2.reshape(H, 1), W3)

    p3 = _agg_call(g3t.reshape(-1), src, dst, ew).reshape(NCHUNK, H, N)
    return _post_call(p3, g3t, dinv, b3.reshape(H, 1), batch2d, Wl,
                      bl.reshape(1, C))


# speedup vs baseline: 1.4641x; 1.0689x over previous
import functools

import jax
import jax.numpy as jnp
from jax import lax
from jax.experimental import pallas as pl
from jax.experimental.pallas import tpu as pltpu
from jax.experimental.pallas import tpu_sc as plsc

N = 10000
E = 320000
D = 128
H = 32
C = 10
G = 64

NTILES = 32
DEG_PER_TILE = E // NTILES
NGROUP = 8
NCHUNK = NTILES // NGROUP
FPG = H // NGROUP
EDGES_PER_CHUNK = E // NCHUNK
SUB = 8000
NSUB = EDGES_PER_CHUNK // SUB

_mesh = plsc.VectorSubcoreMesh(core_axis_name="c", subcore_axis_name="s")

_HIGH = lax.Precision.HIGHEST


def _dot(a, b, dims):
    return lax.dot_general(a, b, (dims, ((), ())),
                           precision=_HIGH, preferred_element_type=jnp.float32)


@functools.partial(
    pl.kernel,
    out_type=jax.ShapeDtypeStruct((NTILES, N), jnp.float32),
    mesh=_mesh,
    compiler_params=pltpu.CompilerParams(needs_layout_passes=False),
    scratch_types=[
        pltpu.VMEM((DEG_PER_TILE,), jnp.int32),
        pltpu.VMEM((DEG_PER_TILE,), jnp.float32),
        pltpu.VMEM((N,), jnp.float32),
    ],
)
def _deg_call(dst_hbm, ew_hbm, out_hbm, dst_v, ew_v, acc_v):
    tid = lax.axis_index("c") * 16 + lax.axis_index("s")
    base = tid * DEG_PER_TILE
    pltpu.sync_copy(dst_hbm.at[pl.ds(base, DEG_PER_TILE)], dst_v)
    pltpu.sync_copy(ew_hbm.at[pl.ds(base, DEG_PER_TILE)], ew_v)

    zero16 = jnp.zeros((16,), jnp.float32)

    @plsc.parallel_loop(0, N // 16, unroll=8)
    def zb(i):
        acc_v[pl.ds(i * 16, 16)] = zero16

    @plsc.parallel_loop(0, DEG_PER_TILE // 16, unroll=4)
    def eb(i):
        d = dst_v[pl.ds(i * 16, 16)]
        w = ew_v[pl.ds(i * 16, 16)]
        plsc.addupdate_scatter(acc_v, [d], w)

    pltpu.sync_copy(acc_v, out_hbm.at[tid])


@functools.partial(
    pl.kernel,
    out_type=jax.ShapeDtypeStruct((NCHUNK * H * N,), jnp.float32),
    mesh=_mesh,
    compiler_params=pltpu.CompilerParams(needs_layout_passes=False),
    scratch_types=[
        pltpu.VMEM((FPG // 2 * N,), jnp.int32),
        pltpu.VMEM((FPG * N,), jnp.float32),
        pltpu.VMEM((2 * SUB,), jnp.int32),
        pltpu.VMEM((2 * SUB,), jnp.int32),
        pltpu.VMEM((2 * SUB,), jnp.float32),
        pltpu.SemaphoreType.DMA,
        pltpu.SemaphoreType.DMA,
        pltpu.SemaphoreType.DMA,
    ],
)
def _agg_call(gp_hbm, src_hbm, dst_hbm, ew_hbm, out_hbm,
              gp_v, acc_v, src_v, dst_v, ew_v, gsem, sem0, sem1):
    tid = lax.axis_index("c") * 16 + lax.axis_index("s")
    grp = tid % NGROUP
    chunk = tid // NGROUP
    npair = FPG // 2

    gt_h = pltpu.async_copy(gp_hbm.at[pl.ds(grp * npair * N, npair * N)],
                            gp_v, gsem)

    def issue(k):
        j = k % 2
        base = chunk * EDGES_PER_CHUNK + k * SUB
        sem = sem0 if j == 0 else sem1
        off = j * SUB
        return (
            pltpu.async_copy(src_hbm.at[pl.ds(base, SUB)],
                             src_v.at[pl.ds(off, SUB)], sem),
            pltpu.async_copy(dst_hbm.at[pl.ds(base, SUB)],
                             dst_v.at[pl.ds(off, SUB)], sem),
            pltpu.async_copy(ew_hbm.at[pl.ds(base, SUB)],
                             ew_v.at[pl.ds(off, SUB)], sem),
        )

    pending = [issue(0), None]

    zero16 = jnp.zeros((16,), jnp.float32)

    @plsc.parallel_loop(0, FPG * N // 16, unroll=8)
    def zb(i):
        acc_v[pl.ds(i * 16, 16)] = zero16

    gt_h.wait()

    for k in range(NSUB):
        j = k % 2
        if k + 1 < NSUB:
            pending[(k + 1) % 2] = issue(k + 1)
        for h in pending[j]:
            h.wait()
        off = j * SUB

        @plsc.parallel_loop(0, SUB // 16, unroll=4)
        def eb(i):
            s = src_v[pl.ds(off + i * 16, 16)]
            d = dst_v[pl.ds(off + i * 16, 16)]
            w = ew_v[pl.ds(off + i * 16, 16)]
            for p in range(npair):
                si = s if p == 0 else s + p * N
                pv = plsc.load_gather(gp_v, [si])
                vlo = plsc.bitcast(pv << 16, jnp.float32)
                vhi = plsc.bitcast(pv & jnp.int32(-65536), jnp.float32)
                dlo = d if p == 0 else d + (2 * p) * N
                plsc.addupdate_scatter(acc_v, [dlo], vlo * w)
                plsc.addupdate_scatter(acc_v, [d + (2 * p + 1) * N], vhi * w)

    out_off = (chunk * H + grp * FPG) * N
    pltpu.sync_copy(acc_v, out_hbm.at[pl.ds(out_off, FPG * N)])


def _pack_pairs(g):
    gb = g.astype(jnp.bfloat16)
    g3 = gb.reshape(H // 2, 2, N)
    lo = lax.bitcast_convert_type(g3[:, 0, :], jnp.uint16).astype(jnp.uint32)
    hi = lax.bitcast_convert_type(g3[:, 1, :], jnp.uint16).astype(jnp.uint32)
    return lax.bitcast_convert_type(lo | (hi << 16), jnp.int32)


def _pre_body(x_ref, w1_ref, degp_ref, g1t_ref, g1p_ref, dinv_ref):
    deg = jnp.sum(degp_ref[...], axis=0, keepdims=True) + 1.0
    dinv = lax.rsqrt(deg)
    dinv_ref[...] = dinv
    h1t = _dot(w1_ref[...], x_ref[...], (((0,), (1,))))
    g1t = h1t * dinv
    g1t_ref[...] = g1t
    g1p_ref[...] = _pack_pairs(g1t)


def _pre_call(x, W1, deg_part):
    return pl.pallas_call(
        _pre_body,
        out_shape=[
            jax.ShapeDtypeStruct((H, N), jnp.float32),
            jax.ShapeDtypeStruct((H // 2, N), jnp.int32),
            jax.ShapeDtypeStruct((1, N), jnp.float32),
        ],
    )(x, W1, deg_part)


def _mid_body(part_ref, gt_ref, dinv_ref, b_ref, w_ref, gnext_ref, gp_ref):
    s = jnp.sum(part_ref[...], axis=0)
    dinv = dinv_ref[...]
    z = dinv * (s + gt_ref[...]) + b_ref[...]
    h = jnp.maximum(z, 0.0)
    a = _dot(w_ref[...], h, (((0,), (0,))))
    gnext = a * dinv
    gnext_ref[...] = gnext
    gp_ref[...] = _pack_pairs(gnext)


def _mid_call(part, gt, dinv, b, W):
    return pl.pallas_call(
        _mid_body,
        out_shape=[
            jax.ShapeDtypeStruct((H, N), jnp.float32),
            jax.ShapeDtypeStruct((H // 2, N), jnp.int32),
        ],
    )(part, gt, dinv, b, W)


def _post_body(part_ref, gt_ref, dinv_ref, b_ref, batch_ref, wl_ref, bl_ref,
               out_ref):
    s = jnp.sum(part_ref[...], axis=0)
    z = dinv_ref[...] * (s + gt_ref[...]) + b_ref[...]
    gid = lax.broadcasted_iota(jnp.int32, (G, N), 0)
    onehot = (batch_ref[...] == gid).astype(jnp.float32)
    sums = _dot(z, onehot, (((1,), (1,))))
    counts = jnp.maximum(jnp.sum(onehot, axis=1), 1.0)
    pooled = sums / counts[None, :]
    out_ref[...] = _dot(pooled, wl_ref[...], (((0,), (0,)))) + bl_ref[...]


def _post_call(part, gt, dinv, b, batch2d, Wl, bl):
    return pl.pallas_call(
        _post_body,
        out_shape=jax.ShapeDtypeStruct((G, C), jnp.float32),
    )(part, gt, dinv, b, batch2d, Wl, bl)


def kernel(x, edge_index, edge_weight, batch, W1, b1, W2, b2, W3, b3, Wl, bl):
    src = edge_index[0].astype(jnp.int32)
    dst = edge_index[1].astype(jnp.int32)
    ew = edge_weight.astype(jnp.float32)
    batch2d = batch.astype(jnp.int32).reshape(1, N)

    deg_part = _deg_call(dst, ew)
    g1t, g1p, dinv = _pre_call(x, W1, deg_part)

    p1 = _agg_call(g1p.reshape(-1), src, dst, ew).reshape(NCHUNK, H, N)
    g2t, g2p = _mid_call(p1, g1t, dinv, b1.reshape(H, 1), W2)

    p2 = _agg_call(g2p.reshape(-1), src, dst, ew).reshape(NCHUNK, H, N)
    g3t, g3p = _mid_call(p2, g2t, dinv, b2.reshape(H, 1), W3)

    p3 = _agg_call(g3p.reshape(-1), src, dst, ew).reshape(NCHUNK, H, N)
    return _post_call(p3, g3t, dinv, b3.reshape(H, 1), batch2d, Wl,
                      bl.reshape(1, C))
